# Initial kernel scaffold; baseline (speedup 1.0000x reference)
import jax
import jax.numpy as jnp
from jax.experimental import pallas as pl


def kernel(x, W, b_conv, anchors):
    raise NotImplementedError("write your pallas kernel here")


# speedup vs baseline: 2.7232x; 2.7232x over previous
import functools

import jax
import jax.numpy as jnp
from jax.experimental import pallas as pl
from jax.experimental.pallas import tpu as pltpu

NA = 3
NUM_CLASSES = 80
NO = 86
TOPK = 100
H = W_GRID = 64
P = H * W_GRID
K_DIM = 192


def _decode_kernel(xp_ref, w_ref, b_ref, anc_ref, score_ref, cls_ref,
                   cx_ref, cy_ref, bw_ref, bh_ref):
    xp = xp_ref[0]
    w = w_ref[...]
    raw = jnp.dot(xp, w, preferred_element_type=jnp.float32) + b_ref[...]
    pidx = jax.lax.broadcasted_iota(jnp.int32, (P, 1), 0)
    gx = (pidx % W_GRID).astype(jnp.float32)
    gy = (pidx // W_GRID).astype(jnp.float32)
    for a in range(NA):
        head = raw[:, a * NO:(a + 1) * NO]
        tx = head[:, 0:1]
        ty = head[:, 1:2]
        tw = head[:, 2:3]
        th = head[:, 3:4]
        obj = jax.nn.sigmoid(head[:, 4:5])
        quality = jax.nn.sigmoid(head[:, 5:6])
        cls_s = jax.nn.sigmoid(head[:, 6:NO])
        scores = (obj * quality) * cls_s
        best = jnp.max(scores, axis=1)
        bcls = jnp.argmax(scores, axis=1)
        cx = (jax.nn.sigmoid(tx) + gx) * (1.0 / W_GRID)
        cy = (jax.nn.sigmoid(ty) + gy) * (1.0 / H)
        bw = anc_ref[a, 0] * jax.nn.softplus(tw)
        bh = anc_ref[a, 1] * jax.nn.softplus(th)
        score_ref[0, a, :] = best
        cls_ref[0, a, :] = bcls.astype(jnp.float32)
        cx_ref[0, a, :] = cx[:, 0]
        cy_ref[0, a, :] = cy[:, 0]
        bw_ref[0, a, :] = bw[:, 0]
        bh_ref[0, a, :] = bh[:, 0]


def _decode(xp, w2, b2, anchors):
    b = xp.shape[0]
    flat = jax.ShapeDtypeStruct((b, NA, P), jnp.float32)
    out_shapes = tuple(flat for _ in range(6))
    out_spec = pl.BlockSpec((1, NA, P), lambda i: (i, 0, 0))
    return pl.pallas_call(
        _decode_kernel,
        grid=(b,),
        in_specs=[
            pl.BlockSpec((1, P, K_DIM), lambda i: (i, 0, 0)),
            pl.BlockSpec((K_DIM, NA * NO), lambda i: (0, 0)),
            pl.BlockSpec((1, NA * NO), lambda i: (0, 0)),
            pl.BlockSpec((NA, 2), lambda i: (0, 0)),
        ],
        out_specs=tuple(out_spec for _ in range(6)),
        out_shape=out_shapes,
    )(xp, w2, b2, anchors)


def kernel(x, W, b_conv, anchors):
    b = x.shape[0]
    xp = x.reshape(b, 3, H, 8, W_GRID, 8).transpose(0, 2, 4, 1, 3, 5)
    xp = xp.reshape(b, P, K_DIM)
    w2 = W.reshape(NA * NO, K_DIM).T
    b2 = b_conv.reshape(1, NA * NO)
    score, cls, cx, cy, bw, bh = _decode(xp, w2, b2, anchors)
    score = score.reshape(b, NA * P)
    top_scores, top_idx = jax.lax.top_k(score, TOPK)
    def g(t):
        return jnp.take_along_axis(t.reshape(b, NA * P), top_idx, axis=1)
    detections = jnp.stack(
        [top_scores, g(cls), g(cx), g(cy), g(bw), g(bh)], axis=-1)
    return detections


# speedup vs baseline: 2.8676x; 1.0530x over previous
import functools

import jax
import jax.numpy as jnp
from jax import lax
from jax.experimental import pallas as pl
from jax.experimental.pallas import tpu as pltpu
from jax.experimental.pallas import tpu_sc as plsc

NA = 3
NUM_CLASSES = 80
NO = 86
TOPK = 100
H = W_GRID = 64
P = H * W_GRID
K_DIM = 192


def _decode_kernel(xp_ref, w_ref, b_ref, anc_ref, score_ref, cls_ref,
                   cx_ref, cy_ref, bw_ref, bh_ref):
    xp = xp_ref[0]
    w = w_ref[...]
    raw = jnp.dot(xp, w, preferred_element_type=jnp.float32) + b_ref[...]
    pidx = jax.lax.broadcasted_iota(jnp.int32, (P, 1), 0)
    gx = (pidx % W_GRID).astype(jnp.float32)
    gy = (pidx // W_GRID).astype(jnp.float32)
    for a in range(NA):
        head = raw[:, a * NO:(a + 1) * NO]
        tx = head[:, 0:1]
        ty = head[:, 1:2]
        tw = head[:, 2:3]
        th = head[:, 3:4]
        obj = jax.nn.sigmoid(head[:, 4:5])
        quality = jax.nn.sigmoid(head[:, 5:6])
        cls_s = jax.nn.sigmoid(head[:, 6:NO])
        scores = (obj * quality) * cls_s
        best = jnp.max(scores, axis=1)
        bcls = jnp.argmax(scores, axis=1)
        cx = (jax.nn.sigmoid(tx) + gx) * (1.0 / W_GRID)
        cy = (jax.nn.sigmoid(ty) + gy) * (1.0 / H)
        bw = anc_ref[a, 0] * jax.nn.softplus(tw)
        bh = anc_ref[a, 1] * jax.nn.softplus(th)
        score_ref[0, a, :] = best
        cls_ref[0, a, :] = bcls.astype(jnp.float32)
        cx_ref[0, a, :] = cx[:, 0]
        cy_ref[0, a, :] = cy[:, 0]
        bw_ref[0, a, :] = bw[:, 0]
        bh_ref[0, a, :] = bh[:, 0]


def _decode(xp, w2, b2, anchors):
    b = xp.shape[0]
    flat = jax.ShapeDtypeStruct((b, NA, P), jnp.float32)
    out_shapes = tuple(flat for _ in range(6))
    out_spec = pl.BlockSpec((1, NA, P), lambda i: (i, 0, 0))
    return pl.pallas_call(
        _decode_kernel,
        grid=(b,),
        in_specs=[
            pl.BlockSpec((1, P, K_DIM), lambda i: (i, 0, 0)),
            pl.BlockSpec((K_DIM, NA * NO), lambda i: (0, 0)),
            pl.BlockSpec((1, NA * NO), lambda i: (0, 0)),
            pl.BlockSpec((NA, 2), lambda i: (0, 0)),
        ],
        out_specs=tuple(out_spec for _ in range(6)),
        out_shape=out_shapes,
    )(xp, w2, b2, anchors)


N = NA * P
NV = N // 16
NSEL = 112
NB = 16


def _select_kernel(score_hbm, cls_hbm, cx_hbm, cy_hbm, bw_hbm, bh_hbm,
                   out_hbm, s_v, a0, a1, a2, a3, a4, hist,
                   selk, seli, sortk, sorti, out_v):
    wid = lax.axis_index("s") * 2 + lax.axis_index("c")

    @pl.when(wid < NB)
    def _():
        b = wid
        pltpu.sync_copy(score_hbm.at[b], s_v)
        pltpu.sync_copy(cls_hbm.at[b], a0)
        pltpu.sync_copy(cx_hbm.at[b], a1)
        pltpu.sync_copy(cy_hbm.at[b], a2)
        pltpu.sync_copy(bw_hbm.at[b], a3)
        pltpu.sync_copy(bh_hbm.at[b], a4)

        lanes = lax.iota(jnp.int32, 16)
        ones = jnp.ones((16,), jnp.int32)
        zeros = jnp.zeros((16,), jnp.int32)

        def radix_level(shift, bmask, pshift, prefix, nb, need):
            for j in range(nb // 16):
                hist[pl.ds(16 * j, 16)] = zeros

            def hbody(i, carry):
                kv = plsc.bitcast(s_v[pl.ds(i * 16, 16)], jnp.int32)
                bucket = (kv >> shift) & bmask
                if pshift is None:
                    plsc.addupdate_scatter(hist, [bucket], ones)
                else:
                    match = (kv >> pshift) == prefix
                    plsc.addupdate_scatter(hist, [bucket], ones, mask=match)
                return carry

            lax.fori_loop(0, NV, hbody, jnp.int32(0))

            def sbody(jc, carry):
                acc, bsel, above_sel = carry
                base = nb - 16 * (jc + 1)
                v = hist[pl.ds(base, 16)]
                rv = jnp.flip(v, 0)
                above = acc + plsc.cumsum(rv) - rv
                m = (above < need) & ((above + rv) >= need)
                bucketnum = base + (15 - lanes)
                bsel = jnp.maximum(bsel, jnp.max(jnp.where(m, bucketnum, -1)))
                above_sel = jnp.maximum(
                    above_sel, jnp.max(jnp.where(m, above, -1)))
                return acc + jnp.sum(v), bsel, above_sel

            _, bsel, above = lax.fori_loop(
                0, nb // 16, sbody,
                (jnp.int32(0), jnp.int32(-1), jnp.int32(-1)))
            return bsel, above

        need0 = jnp.int32(TOPK)
        b0, ab0 = radix_level(23, 0x1FF, None, None, 512, need0)
        need1 = need0 - ab0
        b1, ab1 = radix_level(14, 0x1FF, 23, b0, 512, need1)
        pre2 = (b0 << 9) | b1
        need2 = need1 - ab1
        b2, ab2 = radix_level(5, 0x1FF, 14, pre2, 512, need2)
        pre3 = (pre2 << 9) | b2
        need3 = need2 - ab2
        b3, ab3 = radix_level(0, 0x1F, 5, pre3, 32, need3)
        tkey = (pre3 << 5) | b3
        need_eq = need3 - ab3

        big = jnp.full((16,), 2**30, jnp.int32)
        for v in range(NSEL // 16):
            selk[pl.ds(16 * v, 16)] = jnp.full((16,), -1, jnp.int32)
            seli[pl.ds(16 * v, 16)] = big

        def cbody(i, carry):
            c, c1 = carry
            kv = plsc.bitcast(s_v[pl.ds(i * 16, 16)], jnp.int32)
            ivec = lanes + i * 16
            m2 = kv > tkey
            m1 = kv == tkey
            incl1 = plsc.cumsum(m1.astype(jnp.int32))
            keepeq = m1 & ((c1 + incl1 - 1) < need_eq)
            keep = m2 | keepeq
            inclk = plsc.cumsum(keep.astype(jnp.int32))
            pos = c + inclk - 1
            plsc.store_scatter(selk, [pos], kv, mask=keep)
            plsc.store_scatter(seli, [pos], ivec, mask=keep)
            return (c + jnp.sum(keep.astype(jnp.int32)),
                    c1 + jnp.sum(m1.astype(jnp.int32)))

        lax.fori_loop(0, NV, cbody, (jnp.int32(0), jnp.int32(0)))

        def rbody(r, carry):
            bestk = jnp.full((16,), -1, jnp.int32)
            besti = big
            for v in range(NSEL // 16):
                kvv = selk[pl.ds(16 * v, 16)]
                ivv = seli[pl.ds(16 * v, 16)]
                better = (kvv > bestk) | ((kvv == bestk) & (ivv < besti))
                bestk = jnp.where(better, kvv, bestk)
                besti = jnp.where(better, ivv, besti)
            bk = jnp.max(bestk)
            bi = jnp.min(jnp.where(bestk == bk, besti, big))
            rvec = jnp.full((16,), r, jnp.int32)
            lane0 = lanes == 0
            plsc.store_scatter(sortk, [rvec], jnp.full((16,), bk, jnp.int32),
                               mask=lane0)
            plsc.store_scatter(sorti, [rvec], jnp.full((16,), bi, jnp.int32),
                               mask=lane0)
            for v in range(NSEL // 16):
                kvv = selk[pl.ds(16 * v, 16)]
                ivv = seli[pl.ds(16 * v, 16)]
                m = (kvv == bk) & (ivv == bi)
                selk[pl.ds(16 * v, 16)] = jnp.where(m, -1, kvv)
            return carry

        lax.fori_loop(0, TOPK, rbody, jnp.int32(0))

        for v in range(NSEL // 16):
            kvv = sortk[pl.ds(16 * v, 16)]
            ivv = sorti[pl.ds(16 * v, 16)]
            valid = (lanes + 16 * v) < TOPK
            idx = jnp.where(valid, ivv, 0)
            out_v[0, pl.ds(16 * v, 16)] = plsc.bitcast(kvv, jnp.float32)
            out_v[1, pl.ds(16 * v, 16)] = plsc.load_gather(a0, [idx])
            out_v[2, pl.ds(16 * v, 16)] = plsc.load_gather(a1, [idx])
            out_v[3, pl.ds(16 * v, 16)] = plsc.load_gather(a2, [idx])
            out_v[4, pl.ds(16 * v, 16)] = plsc.load_gather(a3, [idx])
            out_v[5, pl.ds(16 * v, 16)] = plsc.load_gather(a4, [idx])
        pltpu.sync_copy(out_v, out_hbm.at[b])


def _sc_select(score, cls_, cx, cy, bw, bh):
    mesh = plsc.VectorSubcoreMesh(
        core_axis_name="c", subcore_axis_name="s",
        num_cores=2, num_subcores=16)
    f = pl.kernel(
        _select_kernel,
        out_type=jax.ShapeDtypeStruct((NB, 6, NSEL), jnp.float32),
        mesh=mesh,
        compiler_params=pltpu.CompilerParams(needs_layout_passes=False),
        scratch_types=[
            pltpu.VMEM((N,), jnp.float32),
            pltpu.VMEM((N,), jnp.float32),
            pltpu.VMEM((N,), jnp.float32),
            pltpu.VMEM((N,), jnp.float32),
            pltpu.VMEM((N,), jnp.float32),
            pltpu.VMEM((N,), jnp.float32),
            pltpu.VMEM((512,), jnp.int32),
            pltpu.VMEM((NSEL,), jnp.int32),
            pltpu.VMEM((NSEL,), jnp.int32),
            pltpu.VMEM((NSEL,), jnp.int32),
            pltpu.VMEM((NSEL,), jnp.int32),
            pltpu.VMEM((6, NSEL), jnp.float32),
        ],
    )
    return f(score, cls_, cx, cy, bw, bh)


def kernel(x, W, b_conv, anchors):
    b = x.shape[0]
    xp = x.reshape(b, 3, H, 8, W_GRID, 8).transpose(0, 2, 4, 1, 3, 5)
    xp = xp.reshape(b, P, K_DIM)
    w2 = W.reshape(NA * NO, K_DIM).T
    b2 = b_conv.reshape(1, NA * NO)
    score, cls, cx, cy, bw, bh = _decode(xp, w2, b2, anchors)
    det6 = _sc_select(score.reshape(b, N), cls.reshape(b, N),
                      cx.reshape(b, N), cy.reshape(b, N),
                      bw.reshape(b, N), bh.reshape(b, N))
    return det6[:, :, :TOPK].transpose(0, 2, 1)


# speedup vs baseline: 9.8067x; 3.4199x over previous
import functools

import jax
import jax.numpy as jnp
import numpy as np
from jax import lax
from jax.experimental import pallas as pl
from jax.experimental.pallas import tpu as pltpu
from jax.experimental.pallas import tpu_sc as plsc

NA = 3
NUM_CLASSES = 80
NO = 86
TOPK = 100
H = W_GRID = 64
P = H * W_GRID
K_DIM = 192


NCH = 264


def _decode_kernel(xt_ref, w_ref, b_ref, anc_ref, score_ref, cls_ref,
                   cx_ref, cy_ref, bw_ref, bh_ref):
    xt = xt_ref[0]
    w = w_ref[...]
    raw = jnp.dot(w, xt, preferred_element_type=jnp.float32) + b_ref[...]
    lane = jax.lax.broadcasted_iota(jnp.int32, (1, P), 1)
    gx = (lane % W_GRID).astype(jnp.float32)
    gy = (lane // W_GRID).astype(jnp.float32)
    box = raw[240:264, :]
    sg = jax.nn.sigmoid(box)
    sp = jax.nn.softplus(box)
    for a in range(NA):
        cls_logits = raw[80 * a:80 * a + 80, :]
        sb = sg[16 + a:17 + a, :] * sg[20 + a:21 + a, :]
        prod = sb * jax.nn.sigmoid(cls_logits)
        mx = jnp.max(prod, axis=0, keepdims=True)
        sidx = jax.lax.broadcasted_iota(jnp.int32, (80, P), 0)
        cand = jnp.where(prod == mx, sidx, 127)
        bcls = jnp.min(cand, axis=0, keepdims=True).astype(jnp.float32)
        cx = (sg[a:a + 1, :] + gx) * (1.0 / W_GRID)
        cy = (sg[4 + a:5 + a, :] + gy) * (1.0 / H)
        bw = anc_ref[a, 0] * sp[8 + a:9 + a, :]
        bh = anc_ref[a, 1] * sp[12 + a:13 + a, :]
        score_ref[0, a, :] = mx[0]
        cls_ref[0, a, :] = bcls[0]
        cx_ref[0, a, :] = cx[0]
        cy_ref[0, a, :] = cy[0]
        bw_ref[0, a, :] = bw[0]
        bh_ref[0, a, :] = bh[0]


def _decode(xt, w2, b2, anchors):
    b = xt.shape[0]
    flat = jax.ShapeDtypeStruct((b, NA, P), jnp.float32)
    out_shapes = tuple(flat for _ in range(6))
    out_spec = pl.BlockSpec((1, NA, P), lambda i: (i, 0, 0))
    return pl.pallas_call(
        _decode_kernel,
        grid=(b,),
        in_specs=[
            pl.BlockSpec((1, K_DIM, P), lambda i: (i, 0, 0)),
            pl.BlockSpec((NCH, K_DIM), lambda i: (0, 0)),
            pl.BlockSpec((NCH, 1), lambda i: (0, 0)),
            pl.BlockSpec((NA, 2), lambda i: (0, 0)),
        ],
        out_specs=tuple(out_spec for _ in range(6)),
        out_shape=out_shapes,
    )(xt, w2, b2, anchors)


def _permute_head(W, b_conv):
    wr = W.reshape(NA * NO, K_DIM)
    cls_rows = np.array(
        [a * NO + 6 + j for a in range(NA) for j in range(NUM_CLASSES)],
        dtype=np.int32)
    w_cls = wr[cls_rows]
    b_cls = b_conv[cls_rows]
    box_rows = np.zeros((24,), dtype=np.int32)
    box_valid = np.zeros((24,), dtype=np.float32)
    for f in range(6):
        for a in range(NA):
            box_rows[f * 4 + a] = a * NO + f
            box_valid[f * 4 + a] = 1.0
    w_box = wr[box_rows] * box_valid[:, None]
    b_box = b_conv[box_rows] * box_valid
    w2 = jnp.concatenate([w_cls, w_box], axis=0)
    b2 = jnp.concatenate([b_cls, b_box], axis=0).reshape(NCH, 1)
    return w2, b2


N = NA * P
NV = N // 16
NSEL = 112
NB = 16


def _select_kernel(score_hbm, cls_hbm, cx_hbm, cy_hbm, bw_hbm, bh_hbm,
                   out_hbm, s_v, a0, a1, a2, a3, a4, hist,
                   selk, seli, sortk, sorti, out_v):
    wid = lax.axis_index("s") * 2 + lax.axis_index("c")

    @pl.when(wid < NB)
    def _():
        b = wid
        pltpu.sync_copy(score_hbm.at[b], s_v)
        pltpu.sync_copy(cls_hbm.at[b], a0)
        pltpu.sync_copy(cx_hbm.at[b], a1)
        pltpu.sync_copy(cy_hbm.at[b], a2)
        pltpu.sync_copy(bw_hbm.at[b], a3)
        pltpu.sync_copy(bh_hbm.at[b], a4)

        lanes = lax.iota(jnp.int32, 16)
        ones = jnp.ones((16,), jnp.int32)
        zeros = jnp.zeros((16,), jnp.int32)

        def radix_level(shift, bmask, pshift, prefix, nb, need):
            for j in range(nb // 16):
                hist[pl.ds(16 * j, 16)] = zeros

            def hbody(i, carry):
                kv = plsc.bitcast(s_v[pl.ds(i * 16, 16)], jnp.int32)
                bucket = (kv >> shift) & bmask
                if pshift is None:
                    plsc.addupdate_scatter(hist, [bucket], ones)
                else:
                    match = (kv >> pshift) == prefix
                    plsc.addupdate_scatter(hist, [bucket], ones, mask=match)
                return carry

            lax.fori_loop(0, NV, hbody, jnp.int32(0))

            def sbody(jc, carry):
                acc, bsel, above_sel = carry
                base = nb - 16 * (jc + 1)
                v = hist[pl.ds(base, 16)]
                rv = jnp.flip(v, 0)
                above = acc + plsc.cumsum(rv) - rv
                m = (above < need) & ((above + rv) >= need)
                bucketnum = base + (15 - lanes)
                bsel = jnp.maximum(bsel, jnp.max(jnp.where(m, bucketnum, -1)))
                above_sel = jnp.maximum(
                    above_sel, jnp.max(jnp.where(m, above, -1)))
                return acc + jnp.sum(v), bsel, above_sel

            _, bsel, above = lax.fori_loop(
                0, nb // 16, sbody,
                (jnp.int32(0), jnp.int32(-1), jnp.int32(-1)))
            return bsel, above

        need0 = jnp.int32(TOPK)
        b0, ab0 = radix_level(23, 0x1FF, None, None, 512, need0)
        need1 = need0 - ab0
        b1, ab1 = radix_level(14, 0x1FF, 23, b0, 512, need1)
        pre2 = (b0 << 9) | b1
        need2 = need1 - ab1
        b2, ab2 = radix_level(5, 0x1FF, 14, pre2, 512, need2)
        pre3 = (pre2 << 9) | b2
        need3 = need2 - ab2
        b3, ab3 = radix_level(0, 0x1F, 5, pre3, 32, need3)
        tkey = (pre3 << 5) | b3
        need_eq = need3 - ab3

        big = jnp.full((16,), 2**30, jnp.int32)
        for v in range(NSEL // 16):
            selk[pl.ds(16 * v, 16)] = jnp.full((16,), -1, jnp.int32)
            seli[pl.ds(16 * v, 16)] = big

        def cbody(i, carry):
            c, c1 = carry
            kv = plsc.bitcast(s_v[pl.ds(i * 16, 16)], jnp.int32)
            ivec = lanes + i * 16
            m2 = kv > tkey
            m1 = kv == tkey
            incl1 = plsc.cumsum(m1.astype(jnp.int32))
            keepeq = m1 & ((c1 + incl1 - 1) < need_eq)
            keep = m2 | keepeq
            inclk = plsc.cumsum(keep.astype(jnp.int32))
            pos = c + inclk - 1
            plsc.store_scatter(selk, [pos], kv, mask=keep)
            plsc.store_scatter(seli, [pos], ivec, mask=keep)
            return (c + jnp.sum(keep.astype(jnp.int32)),
                    c1 + jnp.sum(m1.astype(jnp.int32)))

        lax.fori_loop(0, NV, cbody, (jnp.int32(0), jnp.int32(0)))

        def rbody(r, carry):
            bestk = jnp.full((16,), -1, jnp.int32)
            besti = big
            for v in range(NSEL // 16):
                kvv = selk[pl.ds(16 * v, 16)]
                ivv = seli[pl.ds(16 * v, 16)]
                better = (kvv > bestk) | ((kvv == bestk) & (ivv < besti))
                bestk = jnp.where(better, kvv, bestk)
                besti = jnp.where(better, ivv, besti)
            bk = jnp.max(bestk)
            bi = jnp.min(jnp.where(bestk == bk, besti, big))
            rvec = jnp.full((16,), r, jnp.int32)
            lane0 = lanes == 0
            plsc.store_scatter(sortk, [rvec], jnp.full((16,), bk, jnp.int32),
                               mask=lane0)
            plsc.store_scatter(sorti, [rvec], jnp.full((16,), bi, jnp.int32),
                               mask=lane0)
            for v in range(NSEL // 16):
                kvv = selk[pl.ds(16 * v, 16)]
                ivv = seli[pl.ds(16 * v, 16)]
                m = (kvv == bk) & (ivv == bi)
                selk[pl.ds(16 * v, 16)] = jnp.where(m, -1, kvv)
            return carry

        lax.fori_loop(0, TOPK, rbody, jnp.int32(0))

        for v in range(NSEL // 16):
            kvv = sortk[pl.ds(16 * v, 16)]
            ivv = sorti[pl.ds(16 * v, 16)]
            valid = (lanes + 16 * v) < TOPK
            idx = jnp.where(valid, ivv, 0)
            out_v[0, pl.ds(16 * v, 16)] = plsc.bitcast(kvv, jnp.float32)
            out_v[1, pl.ds(16 * v, 16)] = plsc.load_gather(a0, [idx])
            out_v[2, pl.ds(16 * v, 16)] = plsc.load_gather(a1, [idx])
            out_v[3, pl.ds(16 * v, 16)] = plsc.load_gather(a2, [idx])
            out_v[4, pl.ds(16 * v, 16)] = plsc.load_gather(a3, [idx])
            out_v[5, pl.ds(16 * v, 16)] = plsc.load_gather(a4, [idx])
        pltpu.sync_copy(out_v, out_hbm.at[b])


def _sc_select(score, cls_, cx, cy, bw, bh):
    mesh = plsc.VectorSubcoreMesh(
        core_axis_name="c", subcore_axis_name="s",
        num_cores=2, num_subcores=16)
    f = pl.kernel(
        _select_kernel,
        out_type=jax.ShapeDtypeStruct((NB, 6, NSEL), jnp.float32),
        mesh=mesh,
        compiler_params=pltpu.CompilerParams(needs_layout_passes=False),
        scratch_types=[
            pltpu.VMEM((N,), jnp.float32),
            pltpu.VMEM((N,), jnp.float32),
            pltpu.VMEM((N,), jnp.float32),
            pltpu.VMEM((N,), jnp.float32),
            pltpu.VMEM((N,), jnp.float32),
            pltpu.VMEM((N,), jnp.float32),
            pltpu.VMEM((512,), jnp.int32),
            pltpu.VMEM((NSEL,), jnp.int32),
            pltpu.VMEM((NSEL,), jnp.int32),
            pltpu.VMEM((NSEL,), jnp.int32),
            pltpu.VMEM((NSEL,), jnp.int32),
            pltpu.VMEM((6, NSEL), jnp.float32),
        ],
    )
    return f(score, cls_, cx, cy, bw, bh)


def kernel(x, W, b_conv, anchors):
    b = x.shape[0]
    xt = x.reshape(b, 3, H, 8, W_GRID, 8).transpose(0, 1, 3, 5, 2, 4)
    xt = xt.reshape(b, K_DIM, P)
    w2, b2 = _permute_head(W, b_conv)
    score, cls, cx, cy, bw, bh = _decode(xt, w2, b2, anchors)
    det6 = _sc_select(score.reshape(b, N), cls.reshape(b, N),
                      cx.reshape(b, N), cy.reshape(b, N),
                      bw.reshape(b, N), bh.reshape(b, N))
    return det6[:, :, :TOPK].transpose(0, 2, 1)


# speedup vs baseline: 9.8375x; 1.0031x over previous
import functools

import jax
import jax.numpy as jnp
import numpy as np
from jax import lax
from jax.experimental import pallas as pl
from jax.experimental.pallas import tpu as pltpu
from jax.experimental.pallas import tpu_sc as plsc

NA = 3
NUM_CLASSES = 80
NO = 86
TOPK = 100
H = W_GRID = 64
P = H * W_GRID
K_DIM = 192


NCH = 264


def _decode_kernel(xt_ref, w_ref, b_ref, anc_ref, score_ref, cls_ref,
                   cx_ref, cy_ref, bw_ref, bh_ref):
    xt = xt_ref[0]
    w = w_ref[...]
    raw = jnp.dot(w, xt, preferred_element_type=jnp.float32) + b_ref[...]
    lane = jax.lax.broadcasted_iota(jnp.int32, (1, P), 1)
    gx = (lane % W_GRID).astype(jnp.float32)
    gy = (lane // W_GRID).astype(jnp.float32)
    box = raw[240:264, :]
    sg = jax.nn.sigmoid(box)
    sp = jax.nn.softplus(box)
    for a in range(NA):
        cls_logits = raw[80 * a:80 * a + 80, :]
        sb = sg[16 + a:17 + a, :] * sg[20 + a:21 + a, :]
        prod = sb * jax.nn.sigmoid(cls_logits)
        mx = jnp.max(prod, axis=0, keepdims=True)
        sidx = jax.lax.broadcasted_iota(jnp.int32, (80, P), 0)
        cand = jnp.where(prod == mx, sidx, 127)
        bcls = jnp.min(cand, axis=0, keepdims=True).astype(jnp.float32)
        cx = (sg[a:a + 1, :] + gx) * (1.0 / W_GRID)
        cy = (sg[4 + a:5 + a, :] + gy) * (1.0 / H)
        bw = anc_ref[a, 0] * sp[8 + a:9 + a, :]
        bh = anc_ref[a, 1] * sp[12 + a:13 + a, :]
        score_ref[0, a, :] = mx[0]
        cls_ref[0, a, :] = bcls[0]
        cx_ref[0, a, :] = cx[0]
        cy_ref[0, a, :] = cy[0]
        bw_ref[0, a, :] = bw[0]
        bh_ref[0, a, :] = bh[0]


def _decode(xt, w2, b2, anchors):
    b = xt.shape[0]
    flat = jax.ShapeDtypeStruct((b, NA, P), jnp.float32)
    out_shapes = tuple(flat for _ in range(6))
    out_spec = pl.BlockSpec((1, NA, P), lambda i: (i, 0, 0))
    return pl.pallas_call(
        _decode_kernel,
        grid=(b,),
        in_specs=[
            pl.BlockSpec((1, K_DIM, P), lambda i: (i, 0, 0)),
            pl.BlockSpec((NCH, K_DIM), lambda i: (0, 0)),
            pl.BlockSpec((NCH, 1), lambda i: (0, 0)),
            pl.BlockSpec((NA, 2), lambda i: (0, 0)),
        ],
        out_specs=tuple(out_spec for _ in range(6)),
        out_shape=out_shapes,
    )(xt, w2, b2, anchors)


def _permute_head(W, b_conv):
    wr = W.reshape(NA * NO, K_DIM)
    cls_rows = np.array(
        [a * NO + 6 + j for a in range(NA) for j in range(NUM_CLASSES)],
        dtype=np.int32)
    w_cls = wr[cls_rows]
    b_cls = b_conv[cls_rows]
    box_rows = np.zeros((24,), dtype=np.int32)
    box_valid = np.zeros((24,), dtype=np.float32)
    for f in range(6):
        for a in range(NA):
            box_rows[f * 4 + a] = a * NO + f
            box_valid[f * 4 + a] = 1.0
    w_box = wr[box_rows] * box_valid[:, None]
    b_box = b_conv[box_rows] * box_valid
    w2 = jnp.concatenate([w_cls, w_box], axis=0)
    b2 = jnp.concatenate([b_cls, b_box], axis=0).reshape(NCH, 1)
    return w2, b2


N = NA * P
NV = N // 16
NSEL = 112
NB = 16


def _select_kernel(score_hbm, cls_hbm, cx_hbm, cy_hbm, bw_hbm, bh_hbm,
                   out_hbm, s_v, a0, a1, a2, a3, a4, hist,
                   selk, seli, sortk, sorti, out_v):
    wid = lax.axis_index("s") * 2 + lax.axis_index("c")

    @pl.when(wid < NB)
    def _():
        b = wid
        pltpu.sync_copy(score_hbm.at[b], s_v)
        pltpu.sync_copy(cls_hbm.at[b], a0)
        pltpu.sync_copy(cx_hbm.at[b], a1)
        pltpu.sync_copy(cy_hbm.at[b], a2)
        pltpu.sync_copy(bw_hbm.at[b], a3)
        pltpu.sync_copy(bh_hbm.at[b], a4)

        def vload(ref, i):
            return ref[i >> 3, pl.ds((i & 7) * 16, 16)]

        lanes = lax.iota(jnp.int32, 16)
        ones = jnp.ones((16,), jnp.int32)
        zeros = jnp.zeros((16,), jnp.int32)

        def radix_level(shift, bmask, pshift, prefix, nb, need):
            for j in range(nb // 16):
                hist[pl.ds(16 * j, 16)] = zeros

            def hbody(i, carry):
                kv = plsc.bitcast(vload(s_v, i), jnp.int32)
                bucket = (kv >> shift) & bmask
                if pshift is None:
                    plsc.addupdate_scatter(hist, [bucket], ones)
                else:
                    match = (kv >> pshift) == prefix
                    plsc.addupdate_scatter(hist, [bucket], ones, mask=match)
                return carry

            lax.fori_loop(0, NV, hbody, jnp.int32(0))

            def sbody(jc, carry):
                acc, bsel, above_sel = carry
                base = nb - 16 * (jc + 1)
                v = hist[pl.ds(base, 16)]
                rv = jnp.flip(v, 0)
                above = acc + plsc.cumsum(rv) - rv
                m = (above < need) & ((above + rv) >= need)
                bucketnum = base + (15 - lanes)
                bsel = jnp.maximum(bsel, jnp.max(jnp.where(m, bucketnum, -1)))
                above_sel = jnp.maximum(
                    above_sel, jnp.max(jnp.where(m, above, -1)))
                return acc + jnp.sum(v), bsel, above_sel

            _, bsel, above = lax.fori_loop(
                0, nb // 16, sbody,
                (jnp.int32(0), jnp.int32(-1), jnp.int32(-1)))
            return bsel, above

        need0 = jnp.int32(TOPK)
        b0, ab0 = radix_level(23, 0x1FF, None, None, 512, need0)
        need1 = need0 - ab0
        b1, ab1 = radix_level(14, 0x1FF, 23, b0, 512, need1)
        pre2 = (b0 << 9) | b1
        need2 = need1 - ab1
        b2, ab2 = radix_level(5, 0x1FF, 14, pre2, 512, need2)
        pre3 = (pre2 << 9) | b2
        need3 = need2 - ab2
        b3, ab3 = radix_level(0, 0x1F, 5, pre3, 32, need3)
        tkey = (pre3 << 5) | b3
        need_eq = need3 - ab3

        big = jnp.full((16,), 2**30, jnp.int32)
        for v in range(NSEL // 16):
            selk[pl.ds(16 * v, 16)] = jnp.full((16,), -1, jnp.int32)
            seli[pl.ds(16 * v, 16)] = big

        def cbody(i, carry):
            c, c1 = carry
            kv = plsc.bitcast(vload(s_v, i), jnp.int32)
            ivec = lanes + i * 16
            m2 = kv > tkey
            m1 = kv == tkey
            incl1 = plsc.cumsum(m1.astype(jnp.int32))
            keepeq = m1 & ((c1 + incl1 - 1) < need_eq)
            keep = m2 | keepeq
            inclk = plsc.cumsum(keep.astype(jnp.int32))
            pos = c + inclk - 1
            plsc.store_scatter(selk, [pos], kv, mask=keep)
            plsc.store_scatter(seli, [pos], ivec, mask=keep)
            return (c + jnp.sum(keep.astype(jnp.int32)),
                    c1 + jnp.sum(m1.astype(jnp.int32)))

        lax.fori_loop(0, NV, cbody, (jnp.int32(0), jnp.int32(0)))

        def rbody(r, carry):
            bestk = jnp.full((16,), -1, jnp.int32)
            besti = big
            for v in range(NSEL // 16):
                kvv = selk[pl.ds(16 * v, 16)]
                ivv = seli[pl.ds(16 * v, 16)]
                better = (kvv > bestk) | ((kvv == bestk) & (ivv < besti))
                bestk = jnp.where(better, kvv, bestk)
                besti = jnp.where(better, ivv, besti)
            bk = jnp.max(bestk)
            bi = jnp.min(jnp.where(bestk == bk, besti, big))
            rvec = jnp.full((16,), r, jnp.int32)
            lane0 = lanes == 0
            plsc.store_scatter(sortk, [rvec], jnp.full((16,), bk, jnp.int32),
                               mask=lane0)
            plsc.store_scatter(sorti, [rvec], jnp.full((16,), bi, jnp.int32),
                               mask=lane0)
            for v in range(NSEL // 16):
                kvv = selk[pl.ds(16 * v, 16)]
                ivv = seli[pl.ds(16 * v, 16)]
                m = (kvv == bk) & (ivv == bi)
                selk[pl.ds(16 * v, 16)] = jnp.where(m, -1, kvv)
            return carry

        lax.fori_loop(0, TOPK, rbody, jnp.int32(0))

        for v in range(NSEL // 16):
            kvv = sortk[pl.ds(16 * v, 16)]
            ivv = sorti[pl.ds(16 * v, 16)]
            valid = (lanes + 16 * v) < TOPK
            idx = jnp.where(valid, ivv, 0)
            ir = idx >> 7
            ic = idx & 127
            out_v[0, pl.ds(16 * v, 16)] = plsc.bitcast(kvv, jnp.float32)
            out_v[1, pl.ds(16 * v, 16)] = plsc.load_gather(a0, [ir, ic])
            out_v[2, pl.ds(16 * v, 16)] = plsc.load_gather(a1, [ir, ic])
            out_v[3, pl.ds(16 * v, 16)] = plsc.load_gather(a2, [ir, ic])
            out_v[4, pl.ds(16 * v, 16)] = plsc.load_gather(a3, [ir, ic])
            out_v[5, pl.ds(16 * v, 16)] = plsc.load_gather(a4, [ir, ic])
        pltpu.sync_copy(out_v, out_hbm.at[b])


def _sc_select(score, cls_, cx, cy, bw, bh):
    mesh = plsc.VectorSubcoreMesh(
        core_axis_name="c", subcore_axis_name="s",
        num_cores=2, num_subcores=16)
    f = pl.kernel(
        _select_kernel,
        out_type=jax.ShapeDtypeStruct((NB, 6, NSEL), jnp.float32),
        mesh=mesh,
        compiler_params=pltpu.CompilerParams(needs_layout_passes=False),
        scratch_types=[
            pltpu.VMEM((N // 128, 128), jnp.float32),
            pltpu.VMEM((N // 128, 128), jnp.float32),
            pltpu.VMEM((N // 128, 128), jnp.float32),
            pltpu.VMEM((N // 128, 128), jnp.float32),
            pltpu.VMEM((N // 128, 128), jnp.float32),
            pltpu.VMEM((N // 128, 128), jnp.float32),
            pltpu.VMEM((512,), jnp.int32),
            pltpu.VMEM((NSEL,), jnp.int32),
            pltpu.VMEM((NSEL,), jnp.int32),
            pltpu.VMEM((NSEL,), jnp.int32),
            pltpu.VMEM((NSEL,), jnp.int32),
            pltpu.VMEM((6, NSEL), jnp.float32),
        ],
    )
    return f(score, cls_, cx, cy, bw, bh)


def kernel(x, W, b_conv, anchors):
    b = x.shape[0]
    xt = x.reshape(b, 3, H, 8, W_GRID, 8).transpose(0, 1, 3, 5, 2, 4)
    xt = xt.reshape(b, K_DIM, P)
    w2, b2 = _permute_head(W, b_conv)
    score, cls, cx, cy, bw, bh = _decode(xt, w2, b2, anchors)
    s3 = (b, N // 128, 128)
    det6 = _sc_select(score.reshape(s3), cls.reshape(s3),
                      cx.reshape(s3), cy.reshape(s3),
                      bw.reshape(s3), bh.reshape(s3))
    return det6[:, :, :TOPK].transpose(0, 2, 1)


# speedup vs baseline: 10.2926x; 1.0463x over previous
import functools

import jax
import jax.numpy as jnp
import numpy as np
from jax import lax
from jax.experimental import pallas as pl
from jax.experimental.pallas import tpu as pltpu
from jax.experimental.pallas import tpu_sc as plsc

NA = 3
NUM_CLASSES = 80
NO = 86
TOPK = 100
H = W_GRID = 64
P = H * W_GRID
K_DIM = 192


NCH = 264


def _decode_kernel(xt_ref, w_ref, b_ref, anc_ref, score_ref, cls_ref,
                   cx_ref, cy_ref, bw_ref, bh_ref):
    xt = xt_ref[0]
    w = w_ref[...]
    raw = jnp.dot(w, xt, preferred_element_type=jnp.float32) + b_ref[...]
    lane = jax.lax.broadcasted_iota(jnp.int32, (1, P), 1)
    gx = (lane % W_GRID).astype(jnp.float32)
    gy = (lane // W_GRID).astype(jnp.float32)
    box = raw[240:264, :]
    sg = jax.nn.sigmoid(box)
    sp = jax.nn.softplus(box)
    for a in range(NA):
        cls_logits = raw[80 * a:80 * a + 80, :]
        sb = sg[16 + a:17 + a, :] * sg[20 + a:21 + a, :]
        prod = sb * jax.nn.sigmoid(cls_logits)
        mx = jnp.max(prod, axis=0, keepdims=True)
        sidx = jax.lax.broadcasted_iota(jnp.int32, (80, P), 0)
        cand = jnp.where(prod == mx, sidx, 127)
        bcls = jnp.min(cand, axis=0, keepdims=True).astype(jnp.float32)
        cx = (sg[a:a + 1, :] + gx) * (1.0 / W_GRID)
        cy = (sg[4 + a:5 + a, :] + gy) * (1.0 / H)
        bw = anc_ref[a, 0] * sp[8 + a:9 + a, :]
        bh = anc_ref[a, 1] * sp[12 + a:13 + a, :]
        rows = pl.ds(32 * a, 32)
        score_ref[0, rows, :] = mx.reshape(32, 128)
        cls_ref[0, rows, :] = bcls.reshape(32, 128)
        cx_ref[0, rows, :] = cx.reshape(32, 128)
        cy_ref[0, rows, :] = cy.reshape(32, 128)
        bw_ref[0, rows, :] = bw.reshape(32, 128)
        bh_ref[0, rows, :] = bh.reshape(32, 128)


def _decode(xt, w2, b2, anchors):
    b = xt.shape[0]
    flat = jax.ShapeDtypeStruct((b, NA * P // 128, 128), jnp.float32)
    out_shapes = tuple(flat for _ in range(6))
    out_spec = pl.BlockSpec((1, NA * P // 128, 128), lambda i: (i, 0, 0))
    return pl.pallas_call(
        _decode_kernel,
        grid=(b,),
        in_specs=[
            pl.BlockSpec((1, K_DIM, P), lambda i: (i, 0, 0)),
            pl.BlockSpec((NCH, K_DIM), lambda i: (0, 0)),
            pl.BlockSpec((NCH, 1), lambda i: (0, 0)),
            pl.BlockSpec((NA, 2), lambda i: (0, 0)),
        ],
        out_specs=tuple(out_spec for _ in range(6)),
        out_shape=out_shapes,
    )(xt, w2, b2, anchors)


def _permute_head(W, b_conv):
    wr = W.reshape(NA * NO, K_DIM)
    cls_rows = np.array(
        [a * NO + 6 + j for a in range(NA) for j in range(NUM_CLASSES)],
        dtype=np.int32)
    w_cls = wr[cls_rows]
    b_cls = b_conv[cls_rows]
    box_rows = np.zeros((24,), dtype=np.int32)
    box_valid = np.zeros((24,), dtype=np.float32)
    for f in range(6):
        for a in range(NA):
            box_rows[f * 4 + a] = a * NO + f
            box_valid[f * 4 + a] = 1.0
    w_box = wr[box_rows] * box_valid[:, None]
    b_box = b_conv[box_rows] * box_valid
    w2 = jnp.concatenate([w_cls, w_box], axis=0)
    b2 = jnp.concatenate([b_cls, b_box], axis=0).reshape(NCH, 1)
    return w2, b2


N = NA * P
NV = N // 16
NSEL = 112
NB = 16


def _select_kernel(score_hbm, cls_hbm, cx_hbm, cy_hbm, bw_hbm, bh_hbm,
                   out_hbm, s_v, a0, a1, a2, a3, a4, hist,
                   selk, seli, sortk, sorti, out_v):
    wid = lax.axis_index("s") * 2 + lax.axis_index("c")

    @pl.when(wid < NB)
    def _():
        b = wid
        pltpu.sync_copy(score_hbm.at[b], s_v)
        pltpu.sync_copy(cls_hbm.at[b], a0)
        pltpu.sync_copy(cx_hbm.at[b], a1)
        pltpu.sync_copy(cy_hbm.at[b], a2)
        pltpu.sync_copy(bw_hbm.at[b], a3)
        pltpu.sync_copy(bh_hbm.at[b], a4)

        def vload(ref, i):
            return ref[i >> 3, pl.ds((i & 7) * 16, 16)]

        lanes = lax.iota(jnp.int32, 16)
        ones = jnp.ones((16,), jnp.int32)
        zeros = jnp.zeros((16,), jnp.int32)

        def radix_level(shift, bmask, pshift, prefix, nb, need):
            for j in range(nb // 16):
                hist[pl.ds(16 * j, 16)] = zeros

            def hbody(i, carry):
                kv = plsc.bitcast(vload(s_v, i), jnp.int32)
                bucket = (kv >> shift) & bmask
                if pshift is None:
                    plsc.addupdate_scatter(hist, [bucket], ones)
                else:
                    match = (kv >> pshift) == prefix
                    plsc.addupdate_scatter(hist, [bucket], ones, mask=match)
                return carry

            lax.fori_loop(0, NV, hbody, jnp.int32(0))

            def sbody(jc, carry):
                acc, bsel, above_sel = carry
                base = nb - 16 * (jc + 1)
                v = hist[pl.ds(base, 16)]
                rv = jnp.flip(v, 0)
                above = acc + plsc.cumsum(rv) - rv
                m = (above < need) & ((above + rv) >= need)
                bucketnum = base + (15 - lanes)
                bsel = jnp.maximum(bsel, jnp.max(jnp.where(m, bucketnum, -1)))
                above_sel = jnp.maximum(
                    above_sel, jnp.max(jnp.where(m, above, -1)))
                return acc + jnp.sum(v), bsel, above_sel

            _, bsel, above = lax.fori_loop(
                0, nb // 16, sbody,
                (jnp.int32(0), jnp.int32(-1), jnp.int32(-1)))
            return bsel, above

        need0 = jnp.int32(TOPK)
        b0, ab0 = radix_level(23, 0x1FF, None, None, 512, need0)
        need1 = need0 - ab0
        b1, ab1 = radix_level(14, 0x1FF, 23, b0, 512, need1)
        pre2 = (b0 << 9) | b1
        need2 = need1 - ab1
        b2, ab2 = radix_level(5, 0x1FF, 14, pre2, 512, need2)
        pre3 = (pre2 << 9) | b2
        need3 = need2 - ab2
        b3, ab3 = radix_level(0, 0x1F, 5, pre3, 32, need3)
        tkey = (pre3 << 5) | b3
        need_eq = need3 - ab3

        big = jnp.full((16,), 2**30, jnp.int32)
        for v in range(NSEL // 16):
            selk[pl.ds(16 * v, 16)] = jnp.full((16,), -1, jnp.int32)
            seli[pl.ds(16 * v, 16)] = big

        def cbody(i, carry):
            c, c1 = carry
            kv = plsc.bitcast(vload(s_v, i), jnp.int32)
            ivec = lanes + i * 16
            m2 = kv > tkey
            m1 = kv == tkey
            incl1 = plsc.cumsum(m1.astype(jnp.int32))
            keepeq = m1 & ((c1 + incl1 - 1) < need_eq)
            keep = m2 | keepeq
            inclk = plsc.cumsum(keep.astype(jnp.int32))
            pos = c + inclk - 1
            plsc.store_scatter(selk, [pos], kv, mask=keep)
            plsc.store_scatter(seli, [pos], ivec, mask=keep)
            return (c + jnp.sum(keep.astype(jnp.int32)),
                    c1 + jnp.sum(m1.astype(jnp.int32)))

        lax.fori_loop(0, NV, cbody, (jnp.int32(0), jnp.int32(0)))

        def rbody(r, carry):
            bestk = jnp.full((16,), -1, jnp.int32)
            besti = big
            for v in range(NSEL // 16):
                kvv = selk[pl.ds(16 * v, 16)]
                ivv = seli[pl.ds(16 * v, 16)]
                better = (kvv > bestk) | ((kvv == bestk) & (ivv < besti))
                bestk = jnp.where(better, kvv, bestk)
                besti = jnp.where(better, ivv, besti)
            bk = jnp.max(bestk)
            bi = jnp.min(jnp.where(bestk == bk, besti, big))
            rvec = jnp.full((16,), r, jnp.int32)
            lane0 = lanes == 0
            plsc.store_scatter(sortk, [rvec], jnp.full((16,), bk, jnp.int32),
                               mask=lane0)
            plsc.store_scatter(sorti, [rvec], jnp.full((16,), bi, jnp.int32),
                               mask=lane0)
            for v in range(NSEL // 16):
                kvv = selk[pl.ds(16 * v, 16)]
                ivv = seli[pl.ds(16 * v, 16)]
                m = (kvv == bk) & (ivv == bi)
                selk[pl.ds(16 * v, 16)] = jnp.where(m, -1, kvv)
            return carry

        lax.fori_loop(0, TOPK, rbody, jnp.int32(0))

        for v in range(NSEL // 16):
            kvv = sortk[pl.ds(16 * v, 16)]
            ivv = sorti[pl.ds(16 * v, 16)]
            valid = (lanes + 16 * v) < TOPK
            idx = jnp.where(valid, ivv, 0)
            ir = idx >> 7
            ic = idx & 127
            out_v[0, pl.ds(16 * v, 16)] = plsc.bitcast(kvv, jnp.float32)
            out_v[1, pl.ds(16 * v, 16)] = plsc.load_gather(a0, [ir, ic])
            out_v[2, pl.ds(16 * v, 16)] = plsc.load_gather(a1, [ir, ic])
            out_v[3, pl.ds(16 * v, 16)] = plsc.load_gather(a2, [ir, ic])
            out_v[4, pl.ds(16 * v, 16)] = plsc.load_gather(a3, [ir, ic])
            out_v[5, pl.ds(16 * v, 16)] = plsc.load_gather(a4, [ir, ic])
        pltpu.sync_copy(out_v, out_hbm.at[b])


def _sc_select(score, cls_, cx, cy, bw, bh):
    mesh = plsc.VectorSubcoreMesh(
        core_axis_name="c", subcore_axis_name="s",
        num_cores=2, num_subcores=16)
    f = pl.kernel(
        _select_kernel,
        out_type=jax.ShapeDtypeStruct((NB, 6, NSEL), jnp.float32),
        mesh=mesh,
        compiler_params=pltpu.CompilerParams(needs_layout_passes=False),
        scratch_types=[
            pltpu.VMEM((N // 128, 128), jnp.float32),
            pltpu.VMEM((N // 128, 128), jnp.float32),
            pltpu.VMEM((N // 128, 128), jnp.float32),
            pltpu.VMEM((N // 128, 128), jnp.float32),
            pltpu.VMEM((N // 128, 128), jnp.float32),
            pltpu.VMEM((N // 128, 128), jnp.float32),
            pltpu.VMEM((512,), jnp.int32),
            pltpu.VMEM((NSEL,), jnp.int32),
            pltpu.VMEM((NSEL,), jnp.int32),
            pltpu.VMEM((NSEL,), jnp.int32),
            pltpu.VMEM((NSEL,), jnp.int32),
            pltpu.VMEM((6, NSEL), jnp.float32),
        ],
    )
    return f(score, cls_, cx, cy, bw, bh)


def kernel(x, W, b_conv, anchors):
    b = x.shape[0]
    xt = x.reshape(b, 3, H, 8, W_GRID, 8).transpose(0, 1, 3, 5, 2, 4)
    xt = xt.reshape(b, K_DIM, P)
    w2, b2 = _permute_head(W, b_conv)
    score, cls, cx, cy, bw, bh = _decode(xt, w2, b2, anchors)
    det6 = _sc_select(score, cls, cx, cy, bw, bh)
    return det6[:, :, :TOPK].transpose(0, 2, 1)
